# Initial kernel scaffold; baseline (speedup 1.0000x reference)
import jax
import jax.numpy as jnp
from jax.experimental import pallas as pl


def kernel(x, rbf, pij, dij, idx_i, idx_j, num_batch, batch_seg, Wres, bres, Wout, bout, Wr, Wproj):
    raise NotImplementedError("write your pallas kernel here")


# speedup vs baseline: 14.1800x; 14.1800x over previous
import functools

import jax
import jax.numpy as jnp
from jax import lax
from jax.experimental import pallas as pl
from jax.experimental.pallas import tpu as pltpu
from jax.experimental.pallas import tpu_sc as plsc

F = 128
R = 16
N = 10000
P = 160000
NL = 16
NREG = F // NL
C_NODES = 40
E_TILE = 32
NCH = -(-N // C_NODES)
NP_ROWS = NCH * C_NODES
P_PAD = 160256
NB_BOUNDS = 272
NW = 32

_DN = (((1,), (1,)), ((), ()))


def _silu(x):
    return x / (1.0 + jnp.exp(-x))


def _dot_t(a, w):
    return lax.dot_general(a, w, _DN, preferred_element_type=jnp.float32)


def _node_kernel(x_ref, Wres_ref, bres_ref, Wout_ref, bout_ref,
                 x1_ref, xx_ref, xsd_ref, q_ref, k_ref, v_ref):
    def rb(z, i):
        w = Wres_ref[i]
        b = bres_ref[i]
        h = _silu(z)
        h = _dot_t(h, w[0]) + b[0:1]
        h = _silu(h)
        h = _dot_t(h, w[1]) + b[1:2]
        return z + h

    def rmlp(z, i, j):
        h = rb(z, i)
        return _dot_t(_silu(h), Wout_ref[j]) + bout_ref[j:j + 1]

    x1 = rb(x_ref[...], 0)
    x1_ref[...] = x1
    xx_ref[...] = rmlp(x1, 2, 0)
    xsd_ref[:, 0:F] = rmlp(x1, 3, 1)
    xsd_ref[:, F:2 * F] = rmlp(x1, 4, 2)
    xsd_ref[:, 2 * F:3 * F] = rmlp(x1, 5, 3)
    q_ref[...] = rmlp(x1, 7, 5)
    k_ref[...] = rmlp(x1, 8, 6)
    v_ref[...] = rmlp(x1, 9, 7)


def _run_node(x, Wres, bres, Wout, bout):
    B = 400
    grid = N // B
    full = lambda a: pl.BlockSpec(a.shape, lambda i: (0,) * a.ndim)
    out = [
        jax.ShapeDtypeStruct((N, F), jnp.float32),
        jax.ShapeDtypeStruct((N, F), jnp.float32),
        jax.ShapeDtypeStruct((N, 3 * F), jnp.float32),
        jax.ShapeDtypeStruct((N, F), jnp.float32),
        jax.ShapeDtypeStruct((N, F), jnp.float32),
        jax.ShapeDtypeStruct((N, F), jnp.float32),
    ]
    return pl.pallas_call(
        _node_kernel,
        grid=(grid,),
        in_specs=[pl.BlockSpec((B, F), lambda i: (i, 0)),
                  full(Wres), full(bres), full(Wout), full(bout)],
        out_specs=[pl.BlockSpec((B, F), lambda i: (i, 0)),
                   pl.BlockSpec((B, F), lambda i: (i, 0)),
                   pl.BlockSpec((B, 3 * F), lambda i: (i, 0)),
                   pl.BlockSpec((B, F), lambda i: (i, 0)),
                   pl.BlockSpec((B, F), lambda i: (i, 0)),
                   pl.BlockSpec((B, F), lambda i: (i, 0))],
        out_shape=out,
    )(x, Wres, bres, Wout, bout)


def _radial_kernel(rbf_ref, Wr_ref, g_ref):
    rbf = rbf_ref[...]
    for t in range(3):
        g_ref[:, t * F:(t + 1) * F] = _dot_t(rbf, Wr_ref[t])


def _run_radial(rbf_pad, Wr):
    B = 512
    grid = P_PAD // B
    return pl.pallas_call(
        _radial_kernel,
        grid=(grid,),
        in_specs=[pl.BlockSpec((B, R), lambda i: (i, 0)),
                  pl.BlockSpec(Wr.shape, lambda i: (0, 0, 0))],
        out_specs=pl.BlockSpec((B, 3 * F), lambda i: (i, 0)),
        out_shape=jax.ShapeDtypeStruct((P_PAD, 3 * F), jnp.float32),
    )(rbf_pad, Wr)


def _edge_kernel(xsd, g, pd, idxi, idxj, bounds,
                 s_out, p_out, d_out,
                 bounds_v, idxi_v, idxj_v, xr_v, g_v, pd_v,
                 acc_s, acc_p, acc_d, sem):
    wid = lax.axis_index("s") * 2 + lax.axis_index("c")
    pltpu.sync_copy(bounds, bounds_v)

    def chunk_body(kk, _):
        c = wid + kk * NW

        @pl.when(c < NCH)
        def _():
            n0 = c * C_NODES
            bv = bounds_v[pl.ds(c, NL)]
            e0 = bv[0]
            e1 = bv[1]
            a = (e0 // 8) * 8
            nt = (e1 - a + E_TILE - 1) // E_TILE

            def zrow(i, _):
                for r in range(NREG):
                    acc_s[i, pl.ds(r * NL, NL)] = jnp.zeros((NL,), jnp.float32)
                for cc in range(3):
                    for r in range(NREG):
                        acc_p[i, cc, pl.ds(r * NL, NL)] = jnp.zeros((NL,), jnp.float32)
                for cc in range(5):
                    for r in range(NREG):
                        acc_d[i, cc, pl.ds(r * NL, NL)] = jnp.zeros((NL,), jnp.float32)
                return 0

            lax.fori_loop(0, C_NODES + 1, zrow, 0)

            def tile_body(t, _):
                base = a + t * E_TILE
                pltpu.sync_copy(idxi.at[pl.ds(base, E_TILE)], idxi_v)
                pltpu.sync_copy(idxj.at[pl.ds(base, E_TILE)], idxj_v)
                pltpu.async_copy(xsd.at[idxj_v], xr_v, sem).wait()
                pltpu.sync_copy(g.at[pl.ds(base, E_TILE)], g_v)
                pltpu.sync_copy(pd.at[pl.ds(base, E_TILE)], pd_v)

                def group_body(gi, _):
                    ivec = idxi_v[pl.ds(gi * NL, NL)]
                    for l in range(NL):
                        e = gi * NL + l
                        iloc = ivec[l] - n0
                        valid = (iloc >= 0) & (iloc < C_NODES)
                        tgt = jnp.where(valid, iloc, C_NODES)
                        pdv = pd_v[e, :]
                        for r in range(NREG):
                            xs = xr_v[e, pl.ds(r * NL, NL)]
                            gs = g_v[e, pl.ds(r * NL, NL)]
                            plsc.addupdate(acc_s.at[tgt, pl.ds(r * NL, NL)], gs * xs)
                        for r in range(NREG):
                            xp = xr_v[e, pl.ds(F + r * NL, NL)]
                            rp = g_v[e, pl.ds(F + r * NL, NL)]
                            tp = rp * xp
                            for cc in range(3):
                                plsc.addupdate(acc_p.at[tgt, cc, pl.ds(r * NL, NL)],
                                               tp * pdv[cc])
                        for r in range(NREG):
                            xd = xr_v[e, pl.ds(2 * F + r * NL, NL)]
                            rd = g_v[e, pl.ds(2 * F + r * NL, NL)]
                            td = rd * xd
                            for cc in range(5):
                                plsc.addupdate(acc_d.at[tgt, cc, pl.ds(r * NL, NL)],
                                               td * pdv[3 + cc])
                    return 0

                lax.fori_loop(0, E_TILE // NL, group_body, 0)
                return 0

            lax.fori_loop(0, nt, tile_body, 0)
            pltpu.sync_copy(acc_s.at[pl.ds(0, C_NODES)], s_out.at[pl.ds(n0, C_NODES)])
            pltpu.sync_copy(acc_p.at[pl.ds(0, C_NODES)], p_out.at[pl.ds(n0, C_NODES)])
            pltpu.sync_copy(acc_d.at[pl.ds(0, C_NODES)], d_out.at[pl.ds(n0, C_NODES)])
        return 0

    lax.fori_loop(0, -(-NCH // NW), chunk_body, 0)


def _run_edges(xsd, g, pd, idxi_pad, idxj_pad, bounds_pad):
    mesh = plsc.VectorSubcoreMesh(core_axis_name="c", subcore_axis_name="s")
    fn = functools.partial(
        pl.kernel,
        mesh=mesh,
        out_type=[jax.ShapeDtypeStruct((NP_ROWS, F), jnp.float32),
                  jax.ShapeDtypeStruct((NP_ROWS, 3, F), jnp.float32),
                  jax.ShapeDtypeStruct((NP_ROWS, 5, F), jnp.float32)],
        scratch_types=[
            pltpu.VMEM((NB_BOUNDS,), jnp.int32),
            pltpu.VMEM((E_TILE,), jnp.int32),
            pltpu.VMEM((E_TILE,), jnp.int32),
            pltpu.VMEM((E_TILE, 3 * F), jnp.float32),
            pltpu.VMEM((E_TILE, 3 * F), jnp.float32),
            pltpu.VMEM((E_TILE, NL), jnp.float32),
            pltpu.VMEM((C_NODES + 1, F), jnp.float32),
            pltpu.VMEM((C_NODES + 1, 3, F), jnp.float32),
            pltpu.VMEM((C_NODES + 1, 5, F), jnp.float32),
            pltpu.SemaphoreType.DMA,
        ],
    )(_edge_kernel)
    return fn(xsd, g, pd, idxi_pad, idxj_pad, bounds_pad)


def _attn_kernel(q_ref, k_ref, v_ref, segq_ref, segk_ref, o_ref):
    q = q_ref[...]
    logits = _dot_t(q, k_ref[...]) * (1.0 / jnp.sqrt(jnp.float32(F)))
    mask = segq_ref[...] == segk_ref[...]
    logits = jnp.where(mask, logits, jnp.float32(-1e9))
    m = jnp.max(logits, axis=1, keepdims=True)
    p = jnp.exp(logits - m)
    s = jnp.sum(p, axis=1, keepdims=True)
    o_ref[...] = lax.dot_general(p, v_ref[...], (((1,), (0,)), ((), ())),
                                 preferred_element_type=jnp.float32) / s


def _run_attn(q, k, v, batch_seg):
    B = 200
    grid = N // B
    segq = batch_seg[:, None]
    segk = batch_seg[None, :]
    return pl.pallas_call(
        _attn_kernel,
        grid=(grid,),
        in_specs=[pl.BlockSpec((B, F), lambda i: (i, 0)),
                  pl.BlockSpec((N, F), lambda i: (0, 0)),
                  pl.BlockSpec((N, F), lambda i: (0, 0)),
                  pl.BlockSpec((B, 1), lambda i: (i, 0)),
                  pl.BlockSpec((1, N), lambda i: (0, 0))],
        out_specs=pl.BlockSpec((B, F), lambda i: (i, 0)),
        out_shape=jax.ShapeDtypeStruct((N, F), jnp.float32),
    )(q, k, v, segq, segk)


def _out_kernel(x1_ref, xx_ref, s_ref, p_ref, d_ref, nl_ref,
                Wres_ref, bres_ref, Wout_ref, bout_ref, Wproj_ref,
                x2_ref, y_ref):
    def rb(z, i):
        w = Wres_ref[i]
        b = bres_ref[i]
        h = _silu(z)
        h = _dot_t(h, w[0]) + b[0:1]
        h = _silu(h)
        h = _dot_t(h, w[1]) + b[1:2]
        return z + h

    def rmlp(z, i, j):
        h = rb(z, i)
        return _dot_t(_silu(h), Wout_ref[j]) + bout_ref[j:j + 1]

    u = xx_ref[...] + s_ref[...]
    for cc in range(3):
        t = _dot_t(p_ref[:, cc, :], Wproj_ref[0])
        u = u + t[:, :F] * t[:, F:]
    for cc in range(5):
        t = _dot_t(d_ref[:, cc, :], Wproj_ref[1])
        u = u + t[:, :F] * t[:, F:]
    loc = rmlp(u, 6, 4)
    z = x1_ref[...] + loc + nl_ref[...]
    x2 = rb(z, 1)
    x2_ref[...] = x2
    y_ref[...] = rmlp(x2, 10, 8)


def _run_out(x1, xx, s_sum, p_sum, d_sum, nl, Wres, bres, Wout, bout, Wproj):
    B = 400
    grid = N // B
    full = lambda a: pl.BlockSpec(a.shape, lambda i: (0,) * a.ndim)
    return pl.pallas_call(
        _out_kernel,
        grid=(grid,),
        in_specs=[pl.BlockSpec((B, F), lambda i: (i, 0)),
                  pl.BlockSpec((B, F), lambda i: (i, 0)),
                  pl.BlockSpec((B, F), lambda i: (i, 0)),
                  pl.BlockSpec((B, 3, F), lambda i: (i, 0, 0)),
                  pl.BlockSpec((B, 5, F), lambda i: (i, 0, 0)),
                  pl.BlockSpec((B, F), lambda i: (i, 0)),
                  full(Wres), full(bres), full(Wout), full(bout), full(Wproj)],
        out_specs=[pl.BlockSpec((B, F), lambda i: (i, 0)),
                   pl.BlockSpec((B, F), lambda i: (i, 0))],
        out_shape=[jax.ShapeDtypeStruct((N, F), jnp.float32),
                   jax.ShapeDtypeStruct((N, F), jnp.float32)],
    )(x1, xx, s_sum, p_sum, d_sum, nl, Wres, bres, Wout, bout, Wproj)


def kernel(x, rbf, pij, dij, idx_i, idx_j, num_batch, batch_seg,
           Wres, bres, Wout, bout, Wr, Wproj):
    x1, xx, xsd, q, k, v = _run_node(x, Wres, bres, Wout, bout)

    rbf_pad = jnp.pad(rbf, ((0, P_PAD - P), (0, 0)))
    g = _run_radial(rbf_pad, Wr)

    pd = jnp.pad(jnp.concatenate([pij, dij], axis=1),
                 ((0, P_PAD - P), (0, NL - 8)))
    idxi_pad = jnp.pad(idx_i, (0, P_PAD - P), constant_values=N)
    idxj_pad = jnp.pad(idx_j, (0, P_PAD - P))
    bounds = jnp.searchsorted(
        idxi_pad, jnp.arange(NCH + 1, dtype=jnp.int32) * C_NODES).astype(jnp.int32)
    bounds_pad = jnp.pad(bounds, (0, NB_BOUNDS - (NCH + 1)),
                         constant_values=P_PAD)

    s_sum, p_sum, d_sum = _run_edges(xsd, g, pd, idxi_pad, idxj_pad, bounds_pad)

    nl = _run_attn(q, k, v, batch_seg)

    x2, y = _run_out(x1, xx, s_sum[:N], p_sum[:N], d_sum[:N], nl,
                     Wres, bres, Wout, bout, Wproj)
    return x2, y


# speedup vs baseline: 15.3582x; 1.0831x over previous
import functools

import jax
import jax.numpy as jnp
from jax import lax
from jax.experimental import pallas as pl
from jax.experimental.pallas import tpu as pltpu
from jax.experimental.pallas import tpu_sc as plsc

F = 128
R = 16
N = 10000
P = 160000
NL = 16
NREG = F // NL
C_NODES = 32
E_TILE = 32
NCH = -(-N // C_NODES)
NP_ROWS = NCH * C_NODES
P_PAD = 160256
NB_BOUNDS = 352
NW = 32

_DN = (((1,), (1,)), ((), ()))


def _silu(x):
    return x / (1.0 + jnp.exp(-x))


def _dot_t(a, w):
    return lax.dot_general(a, w, _DN, preferred_element_type=jnp.float32)


def _node_kernel(x_ref, Wres_ref, bres_ref, Wout_ref, bout_ref,
                 x1_ref, xx_ref, xsd_ref, q_ref, k_ref, v_ref):
    def rb(z, i):
        w = Wres_ref[i]
        b = bres_ref[i]
        h = _silu(z)
        h = _dot_t(h, w[0]) + b[0:1]
        h = _silu(h)
        h = _dot_t(h, w[1]) + b[1:2]
        return z + h

    def rmlp(z, i, j):
        h = rb(z, i)
        return _dot_t(_silu(h), Wout_ref[j]) + bout_ref[j:j + 1]

    x1 = rb(x_ref[...], 0)
    x1_ref[...] = x1
    xx_ref[...] = rmlp(x1, 2, 0)
    xsd_ref[:, 0:F] = rmlp(x1, 3, 1)
    xsd_ref[:, F:2 * F] = rmlp(x1, 4, 2)
    xsd_ref[:, 2 * F:3 * F] = rmlp(x1, 5, 3)
    q_ref[...] = rmlp(x1, 7, 5)
    k_ref[...] = rmlp(x1, 8, 6)
    v_ref[...] = rmlp(x1, 9, 7)


def _run_node(x, Wres, bres, Wout, bout):
    B = 400
    grid = N // B
    full = lambda a: pl.BlockSpec(a.shape, lambda i: (0,) * a.ndim)
    out = [
        jax.ShapeDtypeStruct((N, F), jnp.float32),
        jax.ShapeDtypeStruct((N, F), jnp.float32),
        jax.ShapeDtypeStruct((N, 3 * F), jnp.float32),
        jax.ShapeDtypeStruct((N, F), jnp.float32),
        jax.ShapeDtypeStruct((N, F), jnp.float32),
        jax.ShapeDtypeStruct((N, F), jnp.float32),
    ]
    return pl.pallas_call(
        _node_kernel,
        grid=(grid,),
        in_specs=[pl.BlockSpec((B, F), lambda i: (i, 0)),
                  full(Wres), full(bres), full(Wout), full(bout)],
        out_specs=[pl.BlockSpec((B, F), lambda i: (i, 0)),
                   pl.BlockSpec((B, F), lambda i: (i, 0)),
                   pl.BlockSpec((B, 3 * F), lambda i: (i, 0)),
                   pl.BlockSpec((B, F), lambda i: (i, 0)),
                   pl.BlockSpec((B, F), lambda i: (i, 0)),
                   pl.BlockSpec((B, F), lambda i: (i, 0))],
        out_shape=out,
    )(x, Wres, bres, Wout, bout)


def _radial_kernel(rbf_ref, Wr_ref, g_ref):
    rbf = rbf_ref[...]
    for t in range(3):
        g_ref[:, t * F:(t + 1) * F] = _dot_t(rbf, Wr_ref[t])


def _run_radial(rbf_pad, Wr):
    B = 512
    grid = P_PAD // B
    return pl.pallas_call(
        _radial_kernel,
        grid=(grid,),
        in_specs=[pl.BlockSpec((B, R), lambda i: (i, 0)),
                  pl.BlockSpec(Wr.shape, lambda i: (0, 0, 0))],
        out_specs=pl.BlockSpec((B, 3 * F), lambda i: (i, 0)),
        out_shape=jax.ShapeDtypeStruct((P_PAD, 3 * F), jnp.float32),
    )(rbf_pad, Wr)


def _edge_kernel(xsd, g, pd, idxi, idxj, bounds,
                 s_out, p_out, d_out,
                 bounds_v, idxi_v, idxj_v, xr_v, g_v, pd_v,
                 acc_s, acc_p, acc_d,
                 sem_lin0, sem_lin1, sem_gat0, sem_gat1):
    wid = lax.axis_index("s") * 2 + lax.axis_index("c")
    pltpu.sync_copy(bounds, bounds_v)
    sem_lin = (sem_lin0, sem_lin1)
    sem_gat = (sem_gat0, sem_gat1)

    def fire_lin(b, base):
        pltpu.async_copy(idxi.at[pl.ds(base, E_TILE)], idxi_v.at[b], sem_lin[b])
        pltpu.async_copy(idxj.at[pl.ds(base, E_TILE)], idxj_v.at[b], sem_lin[b])
        pltpu.async_copy(g.at[pl.ds(base, E_TILE)], g_v.at[b], sem_lin[b])
        pltpu.async_copy(pd.at[pl.ds(base, E_TILE)], pd_v.at[b], sem_lin[b])

    def drain_lin(b):
        pltpu.make_async_copy(idxi.at[pl.ds(0, E_TILE)], idxi_v.at[b], sem_lin[b]).wait()
        pltpu.make_async_copy(idxj.at[pl.ds(0, E_TILE)], idxj_v.at[b], sem_lin[b]).wait()
        pltpu.make_async_copy(g.at[pl.ds(0, E_TILE)], g_v.at[b], sem_lin[b]).wait()
        pltpu.make_async_copy(pd.at[pl.ds(0, E_TILE)], pd_v.at[b], sem_lin[b]).wait()

    def fire_gat(b):
        pltpu.async_copy(xsd.at[idxj_v.at[b]], xr_v.at[b], sem_gat[b])

    def drain_gat(b):
        pltpu.make_async_copy(xsd.at[pl.ds(0, E_TILE)], xr_v.at[b], sem_gat[b]).wait()

    def chunk_body(kk, _):
        c = wid + kk * NW

        @pl.when(c < NCH)
        def _():
            n0 = c * C_NODES
            bv = bounds_v[pl.ds(c, NL)]
            e0 = bv[0]
            e1 = bv[1]
            a = (e0 // 8) * 8
            nt = (e1 - a + E_TILE - 1) // E_TILE

            def zrow(i, _):
                for r in range(NREG):
                    acc_s[i, pl.ds(r * NL, NL)] = jnp.zeros((NL,), jnp.float32)
                for cc in range(3):
                    for r in range(NREG):
                        acc_p[i, cc, pl.ds(r * NL, NL)] = jnp.zeros((NL,), jnp.float32)
                for cc in range(5):
                    for r in range(NREG):
                        acc_d[i, cc, pl.ds(r * NL, NL)] = jnp.zeros((NL,), jnp.float32)
                return 0

            @pl.when(nt > 0)
            def _():
                fire_lin(0, a)
                drain_lin(0)
                fire_gat(0)

                @pl.when(nt > 1)
                def _():
                    fire_lin(1, a + E_TILE)

                lax.fori_loop(0, C_NODES + 1, zrow, 0)

                def tile_body(t, _):
                    buf = t % 2

                    @pl.when(t + 1 < nt)
                    def _():
                        for b in range(2):
                            @pl.when((t + 1) % 2 == b)
                            def _():
                                drain_lin(b)
                                fire_gat(b)

                    for b in range(2):
                        @pl.when(buf == b)
                        def _():
                            drain_gat(b)

                    def group_body(gi, _):
                        ivec = idxi_v[buf, pl.ds(gi * NL, NL)]
                        for l in range(NL):
                            e = gi * NL + l
                            iloc = ivec[l] - n0
                            valid = (iloc >= 0) & (iloc < C_NODES)
                            tgt = jnp.where(valid, iloc, C_NODES)
                            pdv = pd_v[buf, e, :]
                            for r in range(NREG):
                                xs = xr_v[buf, e, pl.ds(r * NL, NL)]
                                gs = g_v[buf, e, pl.ds(r * NL, NL)]
                                plsc.addupdate(acc_s.at[tgt, pl.ds(r * NL, NL)],
                                               gs * xs)
                            for r in range(NREG):
                                xp = xr_v[buf, e, pl.ds(F + r * NL, NL)]
                                rp = g_v[buf, e, pl.ds(F + r * NL, NL)]
                                tp = rp * xp
                                for cc in range(3):
                                    plsc.addupdate(
                                        acc_p.at[tgt, cc, pl.ds(r * NL, NL)],
                                        tp * pdv[cc])
                            for r in range(NREG):
                                xd = xr_v[buf, e, pl.ds(2 * F + r * NL, NL)]
                                rd = g_v[buf, e, pl.ds(2 * F + r * NL, NL)]
                                td = rd * xd
                                for cc in range(5):
                                    plsc.addupdate(
                                        acc_d.at[tgt, cc, pl.ds(r * NL, NL)],
                                        td * pdv[3 + cc])
                        return 0

                    lax.fori_loop(0, E_TILE // NL, group_body, 0)

                    @pl.when(t + 2 < nt)
                    def _():
                        for b in range(2):
                            @pl.when(buf == b)
                            def _():
                                fire_lin(b, a + (t + 2) * E_TILE)
                    return 0

                lax.fori_loop(0, nt, tile_body, 0)
                pltpu.sync_copy(acc_s.at[pl.ds(0, C_NODES)],
                                s_out.at[pl.ds(n0, C_NODES)])
                pltpu.sync_copy(acc_p.at[pl.ds(0, C_NODES)],
                                p_out.at[pl.ds(n0, C_NODES)])
                pltpu.sync_copy(acc_d.at[pl.ds(0, C_NODES)],
                                d_out.at[pl.ds(n0, C_NODES)])
        return 0

    lax.fori_loop(0, -(-NCH // NW), chunk_body, 0)


def _run_edges(xsd, g, pd, idxi_pad, idxj_pad, bounds_pad):
    mesh = plsc.VectorSubcoreMesh(core_axis_name="c", subcore_axis_name="s")
    fn = functools.partial(
        pl.kernel,
        mesh=mesh,
        out_type=[jax.ShapeDtypeStruct((NP_ROWS, F), jnp.float32),
                  jax.ShapeDtypeStruct((NP_ROWS, 3, F), jnp.float32),
                  jax.ShapeDtypeStruct((NP_ROWS, 5, F), jnp.float32)],
        scratch_types=[
            pltpu.VMEM((NB_BOUNDS,), jnp.int32),
            pltpu.VMEM((2, E_TILE), jnp.int32),
            pltpu.VMEM((2, E_TILE), jnp.int32),
            pltpu.VMEM((2, E_TILE, 3 * F), jnp.float32),
            pltpu.VMEM((2, E_TILE, 3 * F), jnp.float32),
            pltpu.VMEM((2, E_TILE, NL), jnp.float32),
            pltpu.VMEM((C_NODES + 1, F), jnp.float32),
            pltpu.VMEM((C_NODES + 1, 3, F), jnp.float32),
            pltpu.VMEM((C_NODES + 1, 5, F), jnp.float32),
            pltpu.SemaphoreType.DMA,
            pltpu.SemaphoreType.DMA,
            pltpu.SemaphoreType.DMA,
            pltpu.SemaphoreType.DMA,
        ],
    )(_edge_kernel)
    return fn(xsd, g, pd, idxi_pad, idxj_pad, bounds_pad)


def _attn_kernel(q_ref, k_ref, v_ref, segq_ref, segk_ref, o_ref):
    q = q_ref[...]
    logits = _dot_t(q, k_ref[...]) * (1.0 / jnp.sqrt(jnp.float32(F)))
    mask = segq_ref[...] == segk_ref[...]
    logits = jnp.where(mask, logits, jnp.float32(-1e9))
    m = jnp.max(logits, axis=1, keepdims=True)
    p = jnp.exp(logits - m)
    s = jnp.sum(p, axis=1, keepdims=True)
    o_ref[...] = lax.dot_general(p, v_ref[...], (((1,), (0,)), ((), ())),
                                 preferred_element_type=jnp.float32) / s


def _run_attn(q, k, v, batch_seg):
    B = 200
    grid = N // B
    segq = batch_seg[:, None]
    segk = batch_seg[None, :]
    return pl.pallas_call(
        _attn_kernel,
        grid=(grid,),
        in_specs=[pl.BlockSpec((B, F), lambda i: (i, 0)),
                  pl.BlockSpec((N, F), lambda i: (0, 0)),
                  pl.BlockSpec((N, F), lambda i: (0, 0)),
                  pl.BlockSpec((B, 1), lambda i: (i, 0)),
                  pl.BlockSpec((1, N), lambda i: (0, 0))],
        out_specs=pl.BlockSpec((B, F), lambda i: (i, 0)),
        out_shape=jax.ShapeDtypeStruct((N, F), jnp.float32),
    )(q, k, v, segq, segk)


def _out_kernel(x1_ref, xx_ref, s_ref, p_ref, d_ref, nl_ref,
                Wres_ref, bres_ref, Wout_ref, bout_ref, Wproj_ref,
                x2_ref, y_ref):
    def rb(z, i):
        w = Wres_ref[i]
        b = bres_ref[i]
        h = _silu(z)
        h = _dot_t(h, w[0]) + b[0:1]
        h = _silu(h)
        h = _dot_t(h, w[1]) + b[1:2]
        return z + h

    def rmlp(z, i, j):
        h = rb(z, i)
        return _dot_t(_silu(h), Wout_ref[j]) + bout_ref[j:j + 1]

    u = xx_ref[...] + s_ref[...]
    for cc in range(3):
        t = _dot_t(p_ref[:, cc, :], Wproj_ref[0])
        u = u + t[:, :F] * t[:, F:]
    for cc in range(5):
        t = _dot_t(d_ref[:, cc, :], Wproj_ref[1])
        u = u + t[:, :F] * t[:, F:]
    loc = rmlp(u, 6, 4)
    z = x1_ref[...] + loc + nl_ref[...]
    x2 = rb(z, 1)
    x2_ref[...] = x2
    y_ref[...] = rmlp(x2, 10, 8)


def _run_out(x1, xx, s_sum, p_sum, d_sum, nl, Wres, bres, Wout, bout, Wproj):
    B = 400
    grid = N // B
    full = lambda a: pl.BlockSpec(a.shape, lambda i: (0,) * a.ndim)
    return pl.pallas_call(
        _out_kernel,
        grid=(grid,),
        in_specs=[pl.BlockSpec((B, F), lambda i: (i, 0)),
                  pl.BlockSpec((B, F), lambda i: (i, 0)),
                  pl.BlockSpec((B, F), lambda i: (i, 0)),
                  pl.BlockSpec((B, 3, F), lambda i: (i, 0, 0)),
                  pl.BlockSpec((B, 5, F), lambda i: (i, 0, 0)),
                  pl.BlockSpec((B, F), lambda i: (i, 0)),
                  full(Wres), full(bres), full(Wout), full(bout), full(Wproj)],
        out_specs=[pl.BlockSpec((B, F), lambda i: (i, 0)),
                   pl.BlockSpec((B, F), lambda i: (i, 0))],
        out_shape=[jax.ShapeDtypeStruct((N, F), jnp.float32),
                   jax.ShapeDtypeStruct((N, F), jnp.float32)],
    )(x1, xx, s_sum, p_sum, d_sum, nl, Wres, bres, Wout, bout, Wproj)


def kernel(x, rbf, pij, dij, idx_i, idx_j, num_batch, batch_seg,
           Wres, bres, Wout, bout, Wr, Wproj):
    x1, xx, xsd, q, k, v = _run_node(x, Wres, bres, Wout, bout)

    rbf_pad = jnp.pad(rbf, ((0, P_PAD - P), (0, 0)))
    g = _run_radial(rbf_pad, Wr)

    pd = jnp.pad(jnp.concatenate([pij, dij], axis=1),
                 ((0, P_PAD - P), (0, NL - 8)))
    idxi_pad = jnp.pad(idx_i, (0, P_PAD - P), constant_values=N)
    idxj_pad = jnp.pad(idx_j, (0, P_PAD - P))
    bounds = jnp.searchsorted(
        idxi_pad, jnp.arange(NCH + 1, dtype=jnp.int32) * C_NODES).astype(jnp.int32)
    bounds_pad = jnp.pad(bounds, (0, NB_BOUNDS - (NCH + 1)),
                         constant_values=P_PAD)

    s_sum, p_sum, d_sum = _run_edges(xsd, g, pd, idxi_pad, idxj_pad, bounds_pad)

    nl = _run_attn(q, k, v, batch_seg)

    x2, y = _run_out(x1, xx, s_sum[:N], p_sum[:N], d_sum[:N], nl,
                     Wres, bres, Wout, bout, Wproj)
    return x2, y


# speedup vs baseline: 18.0813x; 1.1773x over previous
import functools

import jax
import jax.numpy as jnp
from jax import lax
from jax.experimental import pallas as pl
from jax.experimental.pallas import tpu as pltpu
from jax.experimental.pallas import tpu_sc as plsc

F = 128
R = 16
N = 10000
P = 160000
NL = 16
NREG = F // NL
C_NODES = 32
E_TILE = 32
NCH = -(-N // C_NODES)
NP_ROWS = NCH * C_NODES
P_PAD = 160256
NB_BOUNDS = 352
NW = 32

GW = 3 * F
_DN = (((1,), (1,)), ((), ()))


def _silu(x):
    return x / (1.0 + jnp.exp(-x))


def _dot_t(a, w):
    return lax.dot_general(a, w, _DN, preferred_element_type=jnp.float32)


def _node_kernel(x_ref, Wres_ref, bres_ref, Wout_ref, bout_ref,
                 x1_ref, xx_ref, xsd_ref, q_ref, k_ref, v_ref):
    def rb(z, i):
        w = Wres_ref[i]
        b = bres_ref[i]
        h = _silu(z)
        h = _dot_t(h, w[0]) + b[0:1]
        h = _silu(h)
        h = _dot_t(h, w[1]) + b[1:2]
        return z + h

    def rmlp(z, i, j):
        h = rb(z, i)
        return _dot_t(_silu(h), Wout_ref[j]) + bout_ref[j:j + 1]

    x1 = rb(x_ref[...], 0)
    x1_ref[...] = x1
    xx_ref[...] = rmlp(x1, 2, 0)
    xsd_ref[:, 0:F] = rmlp(x1, 3, 1)
    xsd_ref[:, F:2 * F] = rmlp(x1, 4, 2)
    xsd_ref[:, 2 * F:3 * F] = rmlp(x1, 5, 3)
    q_ref[...] = rmlp(x1, 7, 5)
    k_ref[...] = rmlp(x1, 8, 6)
    v_ref[...] = rmlp(x1, 9, 7)


def _run_node(x, Wres, bres, Wout, bout):
    B = 400
    grid = N // B
    full = lambda a: pl.BlockSpec(a.shape, lambda i: (0,) * a.ndim)
    out = [
        jax.ShapeDtypeStruct((N, F), jnp.float32),
        jax.ShapeDtypeStruct((N, F), jnp.float32),
        jax.ShapeDtypeStruct((N, 3 * F), jnp.float32),
        jax.ShapeDtypeStruct((N, F), jnp.float32),
        jax.ShapeDtypeStruct((N, F), jnp.float32),
        jax.ShapeDtypeStruct((N, F), jnp.float32),
    ]
    return pl.pallas_call(
        _node_kernel,
        grid=(grid,),
        in_specs=[pl.BlockSpec((B, F), lambda i: (i, 0)),
                  full(Wres), full(bres), full(Wout), full(bout)],
        out_specs=[pl.BlockSpec((B, F), lambda i: (i, 0)),
                   pl.BlockSpec((B, F), lambda i: (i, 0)),
                   pl.BlockSpec((B, 3 * F), lambda i: (i, 0)),
                   pl.BlockSpec((B, F), lambda i: (i, 0)),
                   pl.BlockSpec((B, F), lambda i: (i, 0)),
                   pl.BlockSpec((B, F), lambda i: (i, 0))],
        out_shape=out,
    )(x, Wres, bres, Wout, bout)


def _radial_kernel(rbf_ref, Wr_ref, g_ref):
    rbf = rbf_ref[...]
    for t in range(3):
        g_ref[:, t * F:(t + 1) * F] = _dot_t(rbf, Wr_ref[t])


def _run_radial(rbf_pad, Wr):
    B = 512
    grid = P_PAD // B
    return pl.pallas_call(
        _radial_kernel,
        grid=(grid,),
        in_specs=[pl.BlockSpec((B, R), lambda i: (i, 0)),
                  pl.BlockSpec(Wr.shape, lambda i: (0, 0, 0))],
        out_specs=pl.BlockSpec((B, 3 * F), lambda i: (i, 0)),
        out_shape=jax.ShapeDtypeStruct((P_PAD, 3 * F), jnp.float32),
    )(rbf_pad, Wr)


def _edge_kernel(xsd, g, pd, idxi, idxj, bounds,
                 s_out, p_out, d_out,
                 bounds_v, idxi_v, idxj_v, xr_v, g_v, pd_v,
                 acc_s, acc_p, acc_d,
                 sem_lin0, sem_lin1, sem_gat0, sem_gat1):
    wid = lax.axis_index("s") * 2 + lax.axis_index("c")
    pltpu.sync_copy(bounds, bounds_v)
    sem_lin = (sem_lin0, sem_lin1)
    sem_gat = (sem_gat0, sem_gat1)

    def fire_lin(b, base):
        pltpu.async_copy(idxi.at[pl.ds(base, E_TILE)],
                         idxi_v.at[pl.ds(b * E_TILE, E_TILE)], sem_lin[b])
        pltpu.async_copy(idxj.at[pl.ds(base, E_TILE)],
                         idxj_v.at[pl.ds(b * E_TILE, E_TILE)], sem_lin[b])
        pltpu.async_copy(g.at[pl.ds(base * GW, E_TILE * GW)],
                         g_v.at[pl.ds(b * E_TILE * GW, E_TILE * GW)], sem_lin[b])
        pltpu.async_copy(pd.at[pl.ds(base * NL, E_TILE * NL)],
                         pd_v.at[pl.ds(b * E_TILE * NL, E_TILE * NL)], sem_lin[b])

    def drain_lin(b):
        pltpu.make_async_copy(idxi.at[pl.ds(0, E_TILE)],
                              idxi_v.at[pl.ds(b * E_TILE, E_TILE)], sem_lin[b]).wait()
        pltpu.make_async_copy(idxj.at[pl.ds(0, E_TILE)],
                              idxj_v.at[pl.ds(b * E_TILE, E_TILE)], sem_lin[b]).wait()
        pltpu.make_async_copy(g.at[pl.ds(0, E_TILE * GW)],
                              g_v.at[pl.ds(b * E_TILE * GW, E_TILE * GW)], sem_lin[b]).wait()
        pltpu.make_async_copy(pd.at[pl.ds(0, E_TILE * NL)],
                              pd_v.at[pl.ds(b * E_TILE * NL, E_TILE * NL)], sem_lin[b]).wait()

    def fire_gat(b):
        pltpu.async_copy(xsd.at[idxj_v.at[pl.ds(b * E_TILE, E_TILE)]],
                         xr_v.at[pl.ds(b * E_TILE, E_TILE)], sem_gat[b])

    def drain_gat(b):
        pltpu.make_async_copy(xsd.at[pl.ds(0, E_TILE)],
                              xr_v.at[pl.ds(b * E_TILE, E_TILE)], sem_gat[b]).wait()

    def chunk_body(kk, _):
        c = wid + kk * NW

        @pl.when(c < NCH)
        def _():
            n0 = c * C_NODES
            bv = bounds_v[pl.ds(c, NL)]
            e0 = bv[0]
            e1 = bv[1]
            a = (e0 // 8) * 8
            nt = (e1 - a + E_TILE - 1) // E_TILE

            zv = jnp.zeros((NL,), jnp.float32)

            def zrow(i, _):
                for r in range(NREG):
                    acc_s[pl.ds(i * F + r * NL, NL)] = zv
                for cc in range(3):
                    for r in range(NREG):
                        acc_p[pl.ds(i * 3 * F + cc * F + r * NL, NL)] = zv
                for cc in range(5):
                    for r in range(NREG):
                        acc_d[pl.ds(i * 5 * F + cc * F + r * NL, NL)] = zv
                return 0

            @pl.when(nt > 0)
            def _():
                fire_lin(0, a)
                drain_lin(0)
                fire_gat(0)

                @pl.when(nt > 1)
                def _():
                    fire_lin(1, a + E_TILE)

                lax.fori_loop(0, C_NODES + 1, zrow, 0)

                def tile_body(t, _):
                    buf = t % 2

                    @pl.when(t + 1 < nt)
                    def _():
                        for b in range(2):
                            @pl.when((t + 1) % 2 == b)
                            def _():
                                drain_lin(b)
                                fire_gat(b)

                    for b in range(2):
                        @pl.when(buf == b)
                        def _():
                            drain_gat(b)

                    def group_body(gi, _):
                        ivec = idxi_v[pl.ds(buf * E_TILE + gi * NL, NL)]
                        for l in range(NL):
                            e = buf * E_TILE + gi * NL + l
                            gb = e * GW
                            iloc = ivec[l] - n0
                            valid = (iloc >= 0) & (iloc < C_NODES)
                            tgt = jnp.where(valid, iloc, C_NODES)
                            bs = tgt * F
                            bp = tgt * (3 * F)
                            bd = tgt * (5 * F)
                            pdv = pd_v[pl.ds(e * NL, NL)]
                            xs = [xr_v[e, pl.ds(r * NL, NL)] for r in range(NREG)]
                            xp = [xr_v[e, pl.ds(F + r * NL, NL)] for r in range(NREG)]
                            xd = [xr_v[e, pl.ds(2 * F + r * NL, NL)] for r in range(NREG)]
                            gs = [g_v[pl.ds(gb + r * NL, NL)] for r in range(NREG)]
                            rp = [g_v[pl.ds(gb + F + r * NL, NL)] for r in range(NREG)]
                            rd = [g_v[pl.ds(gb + 2 * F + r * NL, NL)] for r in range(NREG)]
                            ts = [gs[r] * xs[r] for r in range(NREG)]
                            tp = [rp[r] * xp[r] for r in range(NREG)]
                            td = [rd[r] * xd[r] for r in range(NREG)]
                            for r in range(NREG):
                                plsc.addupdate(acc_s.at[pl.ds(bs + r * NL, NL)], ts[r])
                            for cc in range(3):
                                pc = pdv[cc]
                                for r in range(NREG):
                                    plsc.addupdate(
                                        acc_p.at[pl.ds(bp + cc * F + r * NL, NL)],
                                        tp[r] * pc)
                            for cc in range(5):
                                dc = pdv[3 + cc]
                                for r in range(NREG):
                                    plsc.addupdate(
                                        acc_d.at[pl.ds(bd + cc * F + r * NL, NL)],
                                        td[r] * dc)
                        return 0

                    lax.fori_loop(0, E_TILE // NL, group_body, 0)

                    @pl.when(t + 2 < nt)
                    def _():
                        for b in range(2):
                            @pl.when(buf == b)
                            def _():
                                fire_lin(b, a + (t + 2) * E_TILE)
                    return 0

                lax.fori_loop(0, nt, tile_body, 0)
                pltpu.sync_copy(acc_s.at[pl.ds(0, C_NODES * F)],
                                s_out.at[pl.ds(n0 * F, C_NODES * F)])
                pltpu.sync_copy(acc_p.at[pl.ds(0, C_NODES * 3 * F)],
                                p_out.at[pl.ds(n0 * 3 * F, C_NODES * 3 * F)])
                pltpu.sync_copy(acc_d.at[pl.ds(0, C_NODES * 5 * F)],
                                d_out.at[pl.ds(n0 * 5 * F, C_NODES * 5 * F)])
        return 0

    lax.fori_loop(0, -(-NCH // NW), chunk_body, 0)


def _run_edges(xsd, g, pd, idxi_pad, idxj_pad, bounds_pad):
    mesh = plsc.VectorSubcoreMesh(core_axis_name="c", subcore_axis_name="s")
    fn = functools.partial(
        pl.kernel,
        mesh=mesh,
        out_type=[jax.ShapeDtypeStruct((NP_ROWS * F,), jnp.float32),
                  jax.ShapeDtypeStruct((NP_ROWS * 3 * F,), jnp.float32),
                  jax.ShapeDtypeStruct((NP_ROWS * 5 * F,), jnp.float32)],
        scratch_types=[
            pltpu.VMEM((NB_BOUNDS,), jnp.int32),
            pltpu.VMEM((2 * E_TILE,), jnp.int32),
            pltpu.VMEM((2 * E_TILE,), jnp.int32),
            pltpu.VMEM((2 * E_TILE, 3 * F), jnp.float32),
            pltpu.VMEM((2 * E_TILE * 3 * F,), jnp.float32),
            pltpu.VMEM((2 * E_TILE * NL,), jnp.float32),
            pltpu.VMEM(((C_NODES + 1) * F,), jnp.float32),
            pltpu.VMEM(((C_NODES + 1) * 3 * F,), jnp.float32),
            pltpu.VMEM(((C_NODES + 1) * 5 * F,), jnp.float32),
            pltpu.SemaphoreType.DMA,
            pltpu.SemaphoreType.DMA,
            pltpu.SemaphoreType.DMA,
            pltpu.SemaphoreType.DMA,
        ],
    )(_edge_kernel)
    return fn(xsd, g, pd, idxi_pad, idxj_pad, bounds_pad)


def _attn_kernel(q_ref, k_ref, v_ref, segq_ref, segk_ref, o_ref):
    q = q_ref[...]
    logits = _dot_t(q, k_ref[...]) * (1.0 / jnp.sqrt(jnp.float32(F)))
    mask = segq_ref[...] == segk_ref[...]
    logits = jnp.where(mask, logits, jnp.float32(-1e9))
    m = jnp.max(logits, axis=1, keepdims=True)
    p = jnp.exp(logits - m)
    s = jnp.sum(p, axis=1, keepdims=True)
    o_ref[...] = lax.dot_general(p, v_ref[...], (((1,), (0,)), ((), ())),
                                 preferred_element_type=jnp.float32) / s


def _run_attn(q, k, v, batch_seg):
    B = 200
    grid = N // B
    segq = batch_seg[:, None]
    segk = batch_seg[None, :]
    return pl.pallas_call(
        _attn_kernel,
        grid=(grid,),
        in_specs=[pl.BlockSpec((B, F), lambda i: (i, 0)),
                  pl.BlockSpec((N, F), lambda i: (0, 0)),
                  pl.BlockSpec((N, F), lambda i: (0, 0)),
                  pl.BlockSpec((B, 1), lambda i: (i, 0)),
                  pl.BlockSpec((1, N), lambda i: (0, 0))],
        out_specs=pl.BlockSpec((B, F), lambda i: (i, 0)),
        out_shape=jax.ShapeDtypeStruct((N, F), jnp.float32),
    )(q, k, v, segq, segk)


def _out_kernel(x1_ref, xx_ref, s_ref, p_ref, d_ref, nl_ref,
                Wres_ref, bres_ref, Wout_ref, bout_ref, Wproj_ref,
                x2_ref, y_ref):
    def rb(z, i):
        w = Wres_ref[i]
        b = bres_ref[i]
        h = _silu(z)
        h = _dot_t(h, w[0]) + b[0:1]
        h = _silu(h)
        h = _dot_t(h, w[1]) + b[1:2]
        return z + h

    def rmlp(z, i, j):
        h = rb(z, i)
        return _dot_t(_silu(h), Wout_ref[j]) + bout_ref[j:j + 1]

    u = xx_ref[...] + s_ref[...]
    for cc in range(3):
        t = _dot_t(p_ref[:, cc, :], Wproj_ref[0])
        u = u + t[:, :F] * t[:, F:]
    for cc in range(5):
        t = _dot_t(d_ref[:, cc, :], Wproj_ref[1])
        u = u + t[:, :F] * t[:, F:]
    loc = rmlp(u, 6, 4)
    z = x1_ref[...] + loc + nl_ref[...]
    x2 = rb(z, 1)
    x2_ref[...] = x2
    y_ref[...] = rmlp(x2, 10, 8)


def _run_out(x1, xx, s_sum, p_sum, d_sum, nl, Wres, bres, Wout, bout, Wproj):
    B = 400
    grid = N // B
    full = lambda a: pl.BlockSpec(a.shape, lambda i: (0,) * a.ndim)
    return pl.pallas_call(
        _out_kernel,
        grid=(grid,),
        in_specs=[pl.BlockSpec((B, F), lambda i: (i, 0)),
                  pl.BlockSpec((B, F), lambda i: (i, 0)),
                  pl.BlockSpec((B, F), lambda i: (i, 0)),
                  pl.BlockSpec((B, 3, F), lambda i: (i, 0, 0)),
                  pl.BlockSpec((B, 5, F), lambda i: (i, 0, 0)),
                  pl.BlockSpec((B, F), lambda i: (i, 0)),
                  full(Wres), full(bres), full(Wout), full(bout), full(Wproj)],
        out_specs=[pl.BlockSpec((B, F), lambda i: (i, 0)),
                   pl.BlockSpec((B, F), lambda i: (i, 0))],
        out_shape=[jax.ShapeDtypeStruct((N, F), jnp.float32),
                   jax.ShapeDtypeStruct((N, F), jnp.float32)],
    )(x1, xx, s_sum, p_sum, d_sum, nl, Wres, bres, Wout, bout, Wproj)


def kernel(x, rbf, pij, dij, idx_i, idx_j, num_batch, batch_seg,
           Wres, bres, Wout, bout, Wr, Wproj):
    x1, xx, xsd, q, k, v = _run_node(x, Wres, bres, Wout, bout)

    rbf_pad = jnp.pad(rbf, ((0, P_PAD - P), (0, 0)))
    g = _run_radial(rbf_pad, Wr)

    pd = jnp.pad(jnp.concatenate([pij, dij], axis=1),
                 ((0, P_PAD - P), (0, NL - 8))).reshape(-1)
    idxi_pad = jnp.pad(idx_i, (0, P_PAD - P), constant_values=N)
    idxj_pad = jnp.pad(idx_j, (0, P_PAD - P))
    bounds = jnp.searchsorted(
        idxi_pad, jnp.arange(NCH + 1, dtype=jnp.int32) * C_NODES).astype(jnp.int32)
    bounds_pad = jnp.pad(bounds, (0, NB_BOUNDS - (NCH + 1)),
                         constant_values=P_PAD)

    s_sum, p_sum, d_sum = _run_edges(xsd, g.reshape(-1), pd,
                                     idxi_pad, idxj_pad, bounds_pad)
    s_sum = s_sum.reshape(NP_ROWS, F)
    p_sum = p_sum.reshape(NP_ROWS, 3, F)
    d_sum = d_sum.reshape(NP_ROWS, 5, F)

    nl = _run_attn(q, k, v, batch_seg)

    x2, y = _run_out(x1, xx, s_sum[:N], p_sum[:N], d_sum[:N], nl,
                     Wres, bres, Wout, bout, Wproj)
    return x2, y


# speedup vs baseline: 25.6627x; 1.4193x over previous
import functools

import jax
import jax.numpy as jnp
from jax import lax
from jax.experimental import pallas as pl
from jax.experimental.pallas import tpu as pltpu
from jax.experimental.pallas import tpu_sc as plsc

F = 128
R = 16
N = 10000
P = 160000
NL = 16
NREG = F // NL
C_NODES = 32
E_TILE = 32
NCH = -(-N // C_NODES)
NP_ROWS = NCH * C_NODES
P_PAD = 160256
NB_BOUNDS = 352
NW = 32
NL2 = 8

GW = 3 * F
_DN = (((1,), (1,)), ((), ()))


def _silu(x):
    return x / (1.0 + jnp.exp(-x))


def _dot_t(a, w):
    return lax.dot_general(a, w, _DN, preferred_element_type=jnp.float32)


def _node_kernel(x_ref, Wres_ref, bres_ref, Wout_ref, bout_ref,
                 x1_ref, xx_ref, xsd_ref, q_ref, k_ref, v_ref):
    def rb(z, i):
        w = Wres_ref[i]
        b = bres_ref[i]
        h = _silu(z)
        h = _dot_t(h, w[0]) + b[0:1]
        h = _silu(h)
        h = _dot_t(h, w[1]) + b[1:2]
        return z + h

    def rmlp(z, i, j):
        h = rb(z, i)
        return _dot_t(_silu(h), Wout_ref[j]) + bout_ref[j:j + 1]

    x1 = rb(x_ref[...], 0)
    x1_ref[...] = x1
    xx_ref[...] = rmlp(x1, 2, 0)
    xsd_ref[:, 0:F] = rmlp(x1, 3, 1)
    xsd_ref[:, F:2 * F] = rmlp(x1, 4, 2)
    xsd_ref[:, 2 * F:3 * F] = rmlp(x1, 5, 3)
    q_ref[...] = rmlp(x1, 7, 5)
    k_ref[...] = rmlp(x1, 8, 6)
    v_ref[...] = rmlp(x1, 9, 7)


def _run_node(x, Wres, bres, Wout, bout):
    B = 400
    grid = N // B
    full = lambda a: pl.BlockSpec(a.shape, lambda i: (0,) * a.ndim)
    out = [
        jax.ShapeDtypeStruct((N, F), jnp.float32),
        jax.ShapeDtypeStruct((N, F), jnp.float32),
        jax.ShapeDtypeStruct((N, 3 * F), jnp.float32),
        jax.ShapeDtypeStruct((N, F), jnp.float32),
        jax.ShapeDtypeStruct((N, F), jnp.float32),
        jax.ShapeDtypeStruct((N, F), jnp.float32),
    ]
    return pl.pallas_call(
        _node_kernel,
        grid=(grid,),
        in_specs=[pl.BlockSpec((B, F), lambda i: (i, 0)),
                  full(Wres), full(bres), full(Wout), full(bout)],
        out_specs=[pl.BlockSpec((B, F), lambda i: (i, 0)),
                   pl.BlockSpec((B, F), lambda i: (i, 0)),
                   pl.BlockSpec((B, 3 * F), lambda i: (i, 0)),
                   pl.BlockSpec((B, F), lambda i: (i, 0)),
                   pl.BlockSpec((B, F), lambda i: (i, 0)),
                   pl.BlockSpec((B, F), lambda i: (i, 0))],
        out_shape=out,
    )(x, Wres, bres, Wout, bout)


def _radial_kernel(rbf_ref, Wr_ref, g_ref):
    rbf = rbf_ref[...]
    for t in range(3):
        g_ref[:, t * F:(t + 1) * F] = _dot_t(rbf, Wr_ref[t])


def _run_radial(rbf_pad, Wr):
    B = 512
    grid = P_PAD // B
    return pl.pallas_call(
        _radial_kernel,
        grid=(grid,),
        in_specs=[pl.BlockSpec((B, R), lambda i: (i, 0)),
                  pl.BlockSpec(Wr.shape, lambda i: (0, 0, 0))],
        out_specs=pl.BlockSpec((B, 3 * F), lambda i: (i, 0)),
        out_shape=jax.ShapeDtypeStruct((P_PAD, 3 * F), jnp.float32),
    )(rbf_pad, Wr)


def _edge_kernel(xsd, g, pd, idxi, idxj, bounds,
                 s_out, p_out, d_out,
                 bounds_v, idxi_v, idxj_v, xr_v, g_v, pd_v,
                 acc_s, acc_p, acc_d,
                 sem_lin0, sem_lin1, sem_gat0, sem_gat1):
    wid = lax.axis_index("s") * 2 + lax.axis_index("c")
    pltpu.sync_copy(bounds, bounds_v)
    sem_lin = (sem_lin0, sem_lin1)
    sem_gat = (sem_gat0, sem_gat1)

    def fire_lin(b, base):
        pltpu.async_copy(idxi.at[pl.ds(base, E_TILE)],
                         idxi_v.at[pl.ds(b * E_TILE, E_TILE)], sem_lin[b])
        pltpu.async_copy(idxj.at[pl.ds(base, E_TILE)],
                         idxj_v.at[pl.ds(b * E_TILE, E_TILE)], sem_lin[b])
        pltpu.async_copy(g.at[pl.ds(base * GW, E_TILE * GW)],
                         g_v.at[pl.ds(b * E_TILE * GW, E_TILE * GW)], sem_lin[b])
        pltpu.async_copy(pd.at[pl.ds(base * NL, E_TILE * NL)],
                         pd_v.at[pl.ds(b * E_TILE * NL, E_TILE * NL)], sem_lin[b])

    def drain_lin(b):
        pltpu.make_async_copy(idxi.at[pl.ds(0, E_TILE)],
                              idxi_v.at[pl.ds(b * E_TILE, E_TILE)], sem_lin[b]).wait()
        pltpu.make_async_copy(idxj.at[pl.ds(0, E_TILE)],
                              idxj_v.at[pl.ds(b * E_TILE, E_TILE)], sem_lin[b]).wait()
        pltpu.make_async_copy(g.at[pl.ds(0, E_TILE * GW)],
                              g_v.at[pl.ds(b * E_TILE * GW, E_TILE * GW)], sem_lin[b]).wait()
        pltpu.make_async_copy(pd.at[pl.ds(0, E_TILE * NL)],
                              pd_v.at[pl.ds(b * E_TILE * NL, E_TILE * NL)], sem_lin[b]).wait()

    def fire_gat(b):
        pltpu.async_copy(xsd.at[idxj_v.at[pl.ds(b * E_TILE, E_TILE)]],
                         xr_v.at[pl.ds(b * E_TILE, E_TILE)], sem_gat[b])

    def drain_gat(b):
        pltpu.make_async_copy(xsd.at[pl.ds(0, E_TILE)],
                              xr_v.at[pl.ds(b * E_TILE, E_TILE)], sem_gat[b]).wait()

    def chunk_body(kk, _):
        c = wid + kk * NW

        @pl.when(c < NCH)
        def _():
            n0 = c * C_NODES
            bv = bounds_v[pl.ds(c, NL)]
            e0 = bv[0]
            e1 = bv[1]
            a = (e0 // 8) * 8
            nt = (e1 - a + E_TILE - 1) // E_TILE

            zv = jnp.zeros((NL,), jnp.float32)

            def zrow(i, _):
                for r in range(NREG):
                    acc_s[pl.ds(i * F + r * NL, NL)] = zv
                for cc in range(3):
                    for r in range(NREG):
                        acc_p[pl.ds(i * 3 * F + cc * F + r * NL, NL)] = zv
                for cc in range(5):
                    for r in range(NREG):
                        acc_d[pl.ds(i * 5 * F + cc * F + r * NL, NL)] = zv
                return 0

            @pl.when(nt > 0)
            def _():
                fire_lin(0, a)
                drain_lin(0)
                fire_gat(0)

                @pl.when(nt > 1)
                def _():
                    fire_lin(1, a + E_TILE)

                lax.fori_loop(0, C_NODES + 1, zrow, 0)

                def tile_body(t, _):
                    buf = t % 2

                    @pl.when(t + 1 < nt)
                    def _():
                        for b in range(2):
                            @pl.when((t + 1) % 2 == b)
                            def _():
                                drain_lin(b)
                                fire_gat(b)

                    for b in range(2):
                        @pl.when(buf == b)
                        def _():
                            drain_gat(b)

                    def flushA(prev, regs):
                        for r in range(NREG):
                            plsc.addupdate(
                                acc_s.at[pl.ds(prev * F + r * NL, NL)], regs[r])
                        for cc in range(3):
                            for r in range(NREG):
                                plsc.addupdate(
                                    acc_p.at[pl.ds(prev * 3 * F + cc * F + r * NL, NL)],
                                    regs[8 + cc * NREG + r])

                    def gbodyA(gi, carry):
                        prev, regs = carry
                        ivec = idxi_v[pl.ds(buf * E_TILE + gi * NL2, NL)]
                        for l in range(NL2):
                            e = buf * E_TILE + gi * NL2 + l
                            gb = e * GW
                            iloc = ivec[l] - n0
                            valid = (iloc >= 0) & (iloc < C_NODES)
                            tgt = jnp.where(valid, iloc, C_NODES)

                            def fl(ops):
                                pv, rg = ops
                                flushA(pv, rg)
                                return tuple(zv for _ in range(32))

                            regs = lax.cond(tgt != prev, fl,
                                            lambda ops: ops[1], (prev, regs))
                            prev = tgt
                            pdv = pd_v[pl.ds(e * NL, NL)]
                            xs = [xr_v[e, pl.ds(r * NL, NL)] for r in range(NREG)]
                            gs = [g_v[pl.ds(gb + r * NL, NL)] for r in range(NREG)]
                            xp = [xr_v[e, pl.ds(F + r * NL, NL)] for r in range(NREG)]
                            rp = [g_v[pl.ds(gb + F + r * NL, NL)] for r in range(NREG)]
                            tp = [rp[r] * xp[r] for r in range(NREG)]
                            new = ([regs[r] + gs[r] * xs[r] for r in range(NREG)]
                                   + [regs[8 + cc * NREG + r] + tp[r] * pdv[cc]
                                      for cc in range(3) for r in range(NREG)])
                            regs = tuple(new)
                        return prev, regs

                    def flushB(prev, regs):
                        for cc in range(5):
                            for r in range(NREG):
                                plsc.addupdate(
                                    acc_d.at[pl.ds(prev * 5 * F + cc * F + r * NL, NL)],
                                    regs[cc * NREG + r])

                    def gbodyB(gi, carry):
                        prev, regs = carry
                        ivec = idxi_v[pl.ds(buf * E_TILE + gi * NL2, NL)]
                        for l in range(NL2):
                            e = buf * E_TILE + gi * NL2 + l
                            gb = e * GW
                            iloc = ivec[l] - n0
                            valid = (iloc >= 0) & (iloc < C_NODES)
                            tgt = jnp.where(valid, iloc, C_NODES)

                            def fl(ops):
                                pv, rg = ops
                                flushB(pv, rg)
                                return tuple(zv for _ in range(40))

                            regs = lax.cond(tgt != prev, fl,
                                            lambda ops: ops[1], (prev, regs))
                            prev = tgt
                            pdv = pd_v[pl.ds(e * NL, NL)]
                            xd = [xr_v[e, pl.ds(2 * F + r * NL, NL)] for r in range(NREG)]
                            rd = [g_v[pl.ds(gb + 2 * F + r * NL, NL)] for r in range(NREG)]
                            td = [rd[r] * xd[r] for r in range(NREG)]
                            regs = tuple(regs[cc * NREG + r] + td[r] * pdv[3 + cc]
                                         for cc in range(5) for r in range(NREG))
                        return prev, regs

                    initA = (jnp.int32(C_NODES), tuple(zv for _ in range(32)))
                    pA, rA = lax.fori_loop(0, E_TILE // NL2, gbodyA, initA)
                    flushA(pA, rA)
                    initB = (jnp.int32(C_NODES), tuple(zv for _ in range(40)))
                    pB, rB = lax.fori_loop(0, E_TILE // NL2, gbodyB, initB)
                    flushB(pB, rB)

                    @pl.when(t + 2 < nt)
                    def _():
                        for b in range(2):
                            @pl.when(buf == b)
                            def _():
                                fire_lin(b, a + (t + 2) * E_TILE)
                    return 0

                lax.fori_loop(0, nt, tile_body, 0)
                pltpu.sync_copy(acc_s.at[pl.ds(0, C_NODES * F)],
                                s_out.at[pl.ds(n0 * F, C_NODES * F)])
                pltpu.sync_copy(acc_p.at[pl.ds(0, C_NODES * 3 * F)],
                                p_out.at[pl.ds(n0 * 3 * F, C_NODES * 3 * F)])
                pltpu.sync_copy(acc_d.at[pl.ds(0, C_NODES * 5 * F)],
                                d_out.at[pl.ds(n0 * 5 * F, C_NODES * 5 * F)])
        return 0

    lax.fori_loop(0, -(-NCH // NW), chunk_body, 0)


def _run_edges(xsd, g, pd, idxi_pad, idxj_pad, bounds_pad):
    mesh = plsc.VectorSubcoreMesh(core_axis_name="c", subcore_axis_name="s")
    fn = functools.partial(
        pl.kernel,
        mesh=mesh,
        out_type=[jax.ShapeDtypeStruct((NP_ROWS * F,), jnp.float32),
                  jax.ShapeDtypeStruct((NP_ROWS * 3 * F,), jnp.float32),
                  jax.ShapeDtypeStruct((NP_ROWS * 5 * F,), jnp.float32)],
        scratch_types=[
            pltpu.VMEM((NB_BOUNDS,), jnp.int32),
            pltpu.VMEM((2 * E_TILE + NL,), jnp.int32),
            pltpu.VMEM((2 * E_TILE,), jnp.int32),
            pltpu.VMEM((2 * E_TILE, 3 * F), jnp.float32),
            pltpu.VMEM((2 * E_TILE * 3 * F,), jnp.float32),
            pltpu.VMEM((2 * E_TILE * NL,), jnp.float32),
            pltpu.VMEM(((C_NODES + 1) * F,), jnp.float32),
            pltpu.VMEM(((C_NODES + 1) * 3 * F,), jnp.float32),
            pltpu.VMEM(((C_NODES + 1) * 5 * F,), jnp.float32),
            pltpu.SemaphoreType.DMA,
            pltpu.SemaphoreType.DMA,
            pltpu.SemaphoreType.DMA,
            pltpu.SemaphoreType.DMA,
        ],
    )(_edge_kernel)
    return fn(xsd, g, pd, idxi_pad, idxj_pad, bounds_pad)


def _attn_kernel(q_ref, k_ref, v_ref, segq_ref, segk_ref, o_ref):
    q = q_ref[...]
    logits = _dot_t(q, k_ref[...]) * (1.0 / jnp.sqrt(jnp.float32(F)))
    mask = segq_ref[...] == segk_ref[...]
    logits = jnp.where(mask, logits, jnp.float32(-1e9))
    m = jnp.max(logits, axis=1, keepdims=True)
    p = jnp.exp(logits - m)
    s = jnp.sum(p, axis=1, keepdims=True)
    o_ref[...] = lax.dot_general(p, v_ref[...], (((1,), (0,)), ((), ())),
                                 preferred_element_type=jnp.float32) / s


def _run_attn(q, k, v, batch_seg):
    B = 200
    grid = N // B
    segq = batch_seg[:, None]
    segk = batch_seg[None, :]
    return pl.pallas_call(
        _attn_kernel,
        grid=(grid,),
        in_specs=[pl.BlockSpec((B, F), lambda i: (i, 0)),
                  pl.BlockSpec((N, F), lambda i: (0, 0)),
                  pl.BlockSpec((N, F), lambda i: (0, 0)),
                  pl.BlockSpec((B, 1), lambda i: (i, 0)),
                  pl.BlockSpec((1, N), lambda i: (0, 0))],
        out_specs=pl.BlockSpec((B, F), lambda i: (i, 0)),
        out_shape=jax.ShapeDtypeStruct((N, F), jnp.float32),
    )(q, k, v, segq, segk)


def _out_kernel(x1_ref, xx_ref, s_ref, p_ref, d_ref, nl_ref,
                Wres_ref, bres_ref, Wout_ref, bout_ref, Wproj_ref,
                x2_ref, y_ref):
    def rb(z, i):
        w = Wres_ref[i]
        b = bres_ref[i]
        h = _silu(z)
        h = _dot_t(h, w[0]) + b[0:1]
        h = _silu(h)
        h = _dot_t(h, w[1]) + b[1:2]
        return z + h

    def rmlp(z, i, j):
        h = rb(z, i)
        return _dot_t(_silu(h), Wout_ref[j]) + bout_ref[j:j + 1]

    u = xx_ref[...] + s_ref[...]
    for cc in range(3):
        t = _dot_t(p_ref[:, cc, :], Wproj_ref[0])
        u = u + t[:, :F] * t[:, F:]
    for cc in range(5):
        t = _dot_t(d_ref[:, cc, :], Wproj_ref[1])
        u = u + t[:, :F] * t[:, F:]
    loc = rmlp(u, 6, 4)
    z = x1_ref[...] + loc + nl_ref[...]
    x2 = rb(z, 1)
    x2_ref[...] = x2
    y_ref[...] = rmlp(x2, 10, 8)


def _run_out(x1, xx, s_sum, p_sum, d_sum, nl, Wres, bres, Wout, bout, Wproj):
    B = 400
    grid = N // B
    full = lambda a: pl.BlockSpec(a.shape, lambda i: (0,) * a.ndim)
    return pl.pallas_call(
        _out_kernel,
        grid=(grid,),
        in_specs=[pl.BlockSpec((B, F), lambda i: (i, 0)),
                  pl.BlockSpec((B, F), lambda i: (i, 0)),
                  pl.BlockSpec((B, F), lambda i: (i, 0)),
                  pl.BlockSpec((B, 3, F), lambda i: (i, 0, 0)),
                  pl.BlockSpec((B, 5, F), lambda i: (i, 0, 0)),
                  pl.BlockSpec((B, F), lambda i: (i, 0)),
                  full(Wres), full(bres), full(Wout), full(bout), full(Wproj)],
        out_specs=[pl.BlockSpec((B, F), lambda i: (i, 0)),
                   pl.BlockSpec((B, F), lambda i: (i, 0))],
        out_shape=[jax.ShapeDtypeStruct((N, F), jnp.float32),
                   jax.ShapeDtypeStruct((N, F), jnp.float32)],
    )(x1, xx, s_sum, p_sum, d_sum, nl, Wres, bres, Wout, bout, Wproj)


def kernel(x, rbf, pij, dij, idx_i, idx_j, num_batch, batch_seg,
           Wres, bres, Wout, bout, Wr, Wproj):
    x1, xx, xsd, q, k, v = _run_node(x, Wres, bres, Wout, bout)

    rbf_pad = jnp.pad(rbf, ((0, P_PAD - P), (0, 0)))
    g = _run_radial(rbf_pad, Wr)

    pd = jnp.pad(jnp.concatenate([pij, dij], axis=1),
                 ((0, P_PAD - P), (0, NL - 8))).reshape(-1)
    idxi_pad = jnp.pad(idx_i, (0, P_PAD - P), constant_values=N)
    idxj_pad = jnp.pad(idx_j, (0, P_PAD - P))
    bounds = jnp.searchsorted(
        idxi_pad, jnp.arange(NCH + 1, dtype=jnp.int32) * C_NODES).astype(jnp.int32)
    bounds_pad = jnp.pad(bounds, (0, NB_BOUNDS - (NCH + 1)),
                         constant_values=P_PAD)

    s_sum, p_sum, d_sum = _run_edges(xsd, g.reshape(-1), pd,
                                     idxi_pad, idxj_pad, bounds_pad)
    s_sum = s_sum.reshape(NP_ROWS, F)
    p_sum = p_sum.reshape(NP_ROWS, 3, F)
    d_sum = d_sum.reshape(NP_ROWS, 5, F)

    nl = _run_attn(q, k, v, batch_seg)

    x2, y = _run_out(x1, xx, s_sum[:N], p_sum[:N], d_sum[:N], nl,
                     Wres, bres, Wout, bout, Wproj)
    return x2, y
